# baseline (speedup 1.0000x reference)
import functools

import jax
import jax.numpy as jnp
from jax import lax
from jax.experimental import pallas as pl
from jax.experimental.pallas import tpu as pltpu
from jax.experimental.pallas import tpu_sc as plsc

_VOCAB = 1000000
_DIM = 64
_B = 4096
_L = 200

_NC = 2
_NS = 16
_NW = _NC * _NS
_BBLK = _B // _NW

_GDN = lax.GatherDimensionNumbers(offset_dims=(), collapsed_slice_dims=(0,),
                                  start_index_map=(0,))


def _bcast16(vec, i):
    idx = jnp.full((16, 1), i, jnp.int32)
    return lax.gather(vec, idx, _GDN, (1,),
                      mode=lax.GatherScatterMode.PROMISE_IN_BOUNDS)


def _body(wT_hbm, cT_hbm, noun_hbm, class_hbm, special_hbm, out_hbm,
          iw_v, ic_v, cls_v, sp_v, idxb0, idxb1, gbuf0, gbuf1, tbuf0, tbuf1,
          sem_g0, sem_g1, sem_w0, sem_w1):
    wid = lax.axis_index("s") * _NC + lax.axis_index("c")
    b0 = wid * _BBLK

    gbufs = (gbuf0, gbuf1)
    tbufs = (tbuf0, tbuf1)
    idxbs = (idxb0, idxb1)
    sem_g = (sem_g0, sem_g1)
    sem_w = (sem_w0, sem_w1)

    pltpu.sync_copy(wT_hbm.at[:, pl.ds(b0, _BBLK)], iw_v)
    pltpu.sync_copy(cT_hbm.at[:, pl.ds(b0, _BBLK)], ic_v)
    pltpu.sync_copy(class_hbm, cls_v)
    pltpu.sync_copy(special_hbm, sp_v)

    c0g = [cls_v[0, pl.ds(g * 16, 16)] for g in range(4)]
    c1g = [cls_v[1, pl.ds(g * 16, 16)] for g in range(4)]
    spg = [sp_v[0, pl.ds(g * 16, 16)] for g in range(4)]

    for cg in range(4):
        def sp_row(c2, carry, cg=cg):
            bc = _bcast16(spg[cg], c2)
            for bg in range(8):
                tbuf0[cg * 16 + c2, pl.ds(bg * 16, 16)] = bc
            return carry
        lax.fori_loop(0, 16, sp_row, 0)
    pltpu.sync_copy(tbuf0, out_hbm.at[0, :, pl.ds(b0, _BBLK)])

    def prep_idx(l, p):
        for bg in range(8):
            idxbs[p][pl.ds(bg * 16, 16)] = (
                lax.shift_right_logical(iw_v[l, pl.ds(bg * 16, 16)], 1))

    def fire_gather(p):
        pltpu.async_copy(noun_hbm.at[idxbs[p]], gbufs[p], sem_g[p])

    def wait_gather(p):
        pltpu.make_async_copy(noun_hbm.at[idxbs[p]], gbufs[p],
                              sem_g[p]).wait()

    def fire_write(l, p):
        pltpu.async_copy(tbufs[p], out_hbm.at[l + 1, :, pl.ds(b0, _BBLK)],
                         sem_w[p])

    def wait_write(p):
        pltpu.make_async_copy(tbufs[p], out_hbm.at[0, :, pl.ds(b0, _BBLK)],
                              sem_w[p]).wait()

    prep_idx(0, 0)
    fire_gather(0)
    prep_idx(1, 1)
    fire_gather(1)

    def pos_body(l, carry):
        p = l % 2
        iota16 = jnp.arange(16, dtype=jnp.int32)
        bidx = [iota16 + bg * 16 for bg in range(8)]
        masks = [ic_v[l, pl.ds(bg * 16, 16)] != 0 for bg in range(8)]
        coff = [lax.shift_left(iw_v[l, pl.ds(bg * 16, 16)] & 1, 6)
                for bg in range(8)]

        def slot_body(p):
            gbuf = gbufs[p]
            tbuf = tbufs[p]
            wait_gather(p)

            @pl.when(l >= 2)
            def _():
                wait_write(p)

            for cg in range(4):
                def t_row(c2, carry, cg=cg):
                    c = cg * 16 + c2
                    bc0 = _bcast16(c0g[cg], c2)
                    bc1 = _bcast16(c1g[cg], c2)
                    for bg in range(8):
                        cvec = coff[bg] + c
                        v = plsc.load_gather(gbuf, [bidx[bg], cvec])
                        v = v + jnp.where(masks[bg], bc1, bc0)
                        tbuf[c, pl.ds(bg * 16, 16)] = v
                    return carry
                lax.fori_loop(0, 16, t_row, 0)

            fire_write(l, p)

            @pl.when(l + 2 < _L)
            def _():
                prep_idx(l + 2, p)
                fire_gather(p)

        lax.cond(p == 0, lambda: slot_body(0), lambda: slot_body(1))
        return carry

    lax.fori_loop(0, _L, pos_body, 0)
    wait_write(0)
    wait_write(1)


@jax.jit
def _run(words, classes, noun_table, class_table, special_table):
    mesh = plsc.VectorSubcoreMesh(core_axis_name="c", subcore_axis_name="s")
    kern = pl.kernel(
        _body,
        out_type=jax.ShapeDtypeStruct((_L + 1, _DIM, _B), jnp.float32),
        mesh=mesh,
        compiler_params=pltpu.CompilerParams(needs_layout_passes=False,
                                             use_tc_tiling_on_sc=True),
        scratch_types=[
            pltpu.VMEM((_L, _BBLK), jnp.int32),
            pltpu.VMEM((_L, _BBLK), jnp.int32),
            pltpu.VMEM((2, _DIM), jnp.float32),
            pltpu.VMEM((1, _DIM), jnp.float32),
            pltpu.VMEM((_BBLK,), jnp.int32),
            pltpu.VMEM((_BBLK,), jnp.int32),
            pltpu.VMEM((_BBLK, 2 * _DIM), jnp.float32),
            pltpu.VMEM((_BBLK, 2 * _DIM), jnp.float32),
            pltpu.VMEM((_DIM, _BBLK), jnp.float32),
            pltpu.VMEM((_DIM, _BBLK), jnp.float32),
            pltpu.SemaphoreType.DMA,
            pltpu.SemaphoreType.DMA,
            pltpu.SemaphoreType.DMA,
            pltpu.SemaphoreType.DMA,
        ],
    )
    outT = kern(words.T, classes.T,
                noun_table.reshape(_VOCAB // 2, 2 * _DIM),
                class_table, special_table)
    return jnp.transpose(outT, (2, 0, 1))


def kernel(words, classes, noun_table, class_table, special_table):
    return _run(words.astype(jnp.int32), classes.astype(jnp.int32),
                noun_table, class_table, special_table)


# speedup vs baseline: 1.4326x; 1.4326x over previous
import functools

import jax
import jax.numpy as jnp
from jax import lax
from jax.experimental import pallas as pl
from jax.experimental.pallas import tpu as pltpu
from jax.experimental.pallas import tpu_sc as plsc

_VOCAB = 1000000
_DIM = 64
_B = 4096
_L = 200

_NC = 2
_NS = 16
_NW = _NC * _NS
_BBLK = _B // _NW

_GDN = lax.GatherDimensionNumbers(offset_dims=(), collapsed_slice_dims=(0,),
                                  start_index_map=(0,))


def _vperm(vec, idx16):
    return lax.gather(vec, idx16[:, None], _GDN, (1,),
                      mode=lax.GatherScatterMode.PROMISE_IN_BOUNDS)


def _bcast16(vec, i):
    return _vperm(vec, jnp.full((16,), i, jnp.int32))


def _transpose16(rows, masks, perms):
    r = list(rows)
    for si, s in enumerate((1, 2, 4, 8)):
        m, p = masks[si], perms[si]
        for i in range(16):
            if i & s:
                continue
            a, b = r[i], r[i + s]
            r[i] = jnp.where(m, _vperm(b, p), a)
            r[i + s] = jnp.where(m, b, _vperm(a, p))
    return r


def _body(wT_hbm, cT_hbm, noun_hbm, class_hbm, special_hbm, out_hbm,
          iw_v, ic_v, cls_v, sp_v, idxb0, idxb1, gbuf0, gbuf1, tbuf0, tbuf1,
          slab0, slab1, sem_g0, sem_g1, sem_w0, sem_w1):
    wid = lax.axis_index("s") * _NC + lax.axis_index("c")
    b0 = wid * _BBLK

    gbufs = (gbuf0, gbuf1)
    tbufs = (tbuf0, tbuf1)
    idxbs = (idxb0, idxb1)
    sem_g = (sem_g0, sem_g1)
    sem_w = (sem_w0, sem_w1)

    pltpu.sync_copy(wT_hbm.at[:, pl.ds(b0, _BBLK)], iw_v)
    pltpu.sync_copy(cT_hbm.at[:, pl.ds(b0, _BBLK)], ic_v)
    pltpu.sync_copy(class_hbm, cls_v)
    pltpu.sync_copy(special_hbm, sp_v)

    for cg in range(4):
        c0grp = cls_v[0, pl.ds(cg * 16, 16)]
        c1grp = cls_v[1, pl.ds(cg * 16, 16)]
        spgrp = sp_v[0, pl.ds(cg * 16, 16)]

        def slab_row(c2, carry, cg=cg, c0grp=c0grp, c1grp=c1grp, spgrp=spgrp):
            b0v = _bcast16(c0grp, c2)
            b1v = _bcast16(c1grp, c2)
            bsv = _bcast16(spgrp, c2)
            for bg in range(8):
                slab0[cg * 16 + c2, pl.ds(bg * 16, 16)] = b0v
                slab1[cg * 16 + c2, pl.ds(bg * 16, 16)] = b1v
                tbuf0[cg * 16 + c2, pl.ds(bg * 16, 16)] = bsv
            return carry

        lax.fori_loop(0, 16, slab_row, 0)
    pltpu.sync_copy(tbuf0, out_hbm.at[0, :, pl.ds(b0, _BBLK)])

    def prep_idx(l, p):
        for bg in range(8):
            idxbs[p][pl.ds(bg * 16, 16)] = (
                lax.shift_right_logical(iw_v[l, pl.ds(bg * 16, 16)], 1))

    def fire_gather(p):
        pltpu.async_copy(noun_hbm.at[idxbs[p]], gbufs[p], sem_g[p])

    def wait_gather(p):
        pltpu.make_async_copy(noun_hbm.at[idxbs[p]], gbufs[p],
                              sem_g[p]).wait()

    def fire_write(l, p):
        pltpu.async_copy(tbufs[p], out_hbm.at[l + 1, :, pl.ds(b0, _BBLK)],
                         sem_w[p])

    def wait_write(p):
        pltpu.make_async_copy(tbufs[p], out_hbm.at[0, :, pl.ds(b0, _BBLK)],
                              sem_w[p]).wait()

    prep_idx(0, 0)
    fire_gather(0)
    prep_idx(1, 1)
    fire_gather(1)

    iota16 = jnp.arange(16, dtype=jnp.int32)
    tmasks = [(iota16 & s) != 0 for s in (1, 2, 4, 8)]
    tperms = [iota16 ^ s for s in (1, 2, 4, 8)]

    def pos_body(l, carry):
        p = l % 2

        def slot_body(p):
            gbuf = gbufs[p]
            tbuf = tbufs[p]
            wait_gather(p)

            @pl.when(l >= 2)
            def _():
                wait_write(p)

            def bg_body(bg, carry):
                bb = bg * 16
                pv = iw_v[l, pl.ds(bb, 16)] & 1
                pm = [_bcast16(pv, i) != 0 for i in range(16)]
                clsm = ic_v[l, pl.ds(bb, 16)] != 0

                def cg_body(cg, carry):
                    cb = cg * 16
                    rows = []
                    for i in range(16):
                        lo = gbuf[bb + i, pl.ds(cb, 16)]
                        hi = gbuf[bb + i, pl.ds(_DIM + cb, 16)]
                        rows.append(jnp.where(pm[i], hi, lo))
                    t = _transpose16(rows, tmasks, tperms)
                    for j in range(16):
                        c = cb + j
                        v0 = slab0[c, pl.ds(bb, 16)]
                        v1 = slab1[c, pl.ds(bb, 16)]
                        tbuf[c, pl.ds(bb, 16)] = (
                            t[j] + jnp.where(clsm, v1, v0))
                    return carry

                lax.fori_loop(0, 4, cg_body, 0)
                return carry

            lax.fori_loop(0, 8, bg_body, 0)
            fire_write(l, p)

            @pl.when(l + 2 < _L)
            def _():
                prep_idx(l + 2, p)
                fire_gather(p)

        lax.cond(p == 0, lambda: slot_body(0), lambda: slot_body(1))
        return carry

    lax.fori_loop(0, _L, pos_body, 0)
    wait_write(0)
    wait_write(1)


@jax.jit
def _run(words, classes, noun_table, class_table, special_table):
    mesh = plsc.VectorSubcoreMesh(core_axis_name="c", subcore_axis_name="s")
    kern = pl.kernel(
        _body,
        out_type=jax.ShapeDtypeStruct((_L + 1, _DIM, _B), jnp.float32),
        mesh=mesh,
        compiler_params=pltpu.CompilerParams(needs_layout_passes=False,
                                             use_tc_tiling_on_sc=True),
        scratch_types=[
            pltpu.VMEM((_L, _BBLK), jnp.int32),
            pltpu.VMEM((_L, _BBLK), jnp.int32),
            pltpu.VMEM((2, _DIM), jnp.float32),
            pltpu.VMEM((1, _DIM), jnp.float32),
            pltpu.VMEM((_BBLK,), jnp.int32),
            pltpu.VMEM((_BBLK,), jnp.int32),
            pltpu.VMEM((_BBLK, 2 * _DIM), jnp.float32),
            pltpu.VMEM((_BBLK, 2 * _DIM), jnp.float32),
            pltpu.VMEM((_DIM, _BBLK), jnp.float32),
            pltpu.VMEM((_DIM, _BBLK), jnp.float32),
            pltpu.VMEM((_DIM, _BBLK), jnp.float32),
            pltpu.VMEM((_DIM, _BBLK), jnp.float32),
            pltpu.SemaphoreType.DMA,
            pltpu.SemaphoreType.DMA,
            pltpu.SemaphoreType.DMA,
            pltpu.SemaphoreType.DMA,
        ],
    )
    outT = kern(words.T, classes.T,
                noun_table.reshape(_VOCAB // 2, 2 * _DIM),
                class_table, special_table)
    return jnp.transpose(outT, (2, 0, 1))


def kernel(words, classes, noun_table, class_table, special_table):
    return _run(words.astype(jnp.int32), classes.astype(jnp.int32),
                noun_table, class_table, special_table)
